# Initial kernel scaffold; baseline (speedup 1.0000x reference)
import jax
import jax.numpy as jnp
from jax.experimental import pallas as pl


def kernel(x, edge_index, W1, b1, g1, be1, W2, b2, g2, be2, W3, b3, g3, be3, W4, b4):
    raise NotImplementedError("write your pallas kernel here")


# speedup vs baseline: 32.2976x; 32.2976x over previous
import functools

import jax
import jax.numpy as jnp
from jax import lax
from jax.experimental import pallas as pl
from jax.experimental.pallas import tpu as pltpu
from jax.experimental.pallas import tpu_sc as plsc

N = 10000
F = 128
H = 64
NC = 2
NS = 16
NW = NC * NS
CH = 128
NCH = 80
EPAD = NW * NCH * CH
NPAD = 10240
RPT = NPAD // NS
RB = 512
NBLK = NPAD // RB
N2 = NPAD // 2
PB = RB // 2

_Z16 = functools.partial(jnp.zeros, (16,), jnp.float32)


def _mesh():
    return plsc.VectorSubcoreMesh(
        core_axis_name="c", subcore_axis_name="s",
        num_cores=NC, num_subcores=NS)


_SC_PARAMS = pltpu.CompilerParams(use_tc_tiling_on_sc=False)


@functools.cache
def _build_deg_sc():
    return functools.partial(
        pl.kernel,
        out_type=jax.ShapeDtypeStruct((NC, NPAD), jnp.float32),
        mesh=_mesh(),
        compiler_params=_SC_PARAMS,
        scratch_types=[
            pltpu.VMEM((NCH, CH), jnp.int32),
            pltpu.VMEM((CH,), jnp.float32),
            pltpu.VMEM((RPT,), jnp.float32),
            pltpu.VMEM_SHARED((NPAD,), jnp.float32),
        ],
    )(_deg_sc_body)


def _deg_sc_body(dstw, out, dst_v, ones_v, zbuf, acc):
    cid = lax.axis_index("c")
    sid = lax.axis_index("s")
    wid = cid * NS + sid
    pltpu.sync_copy(dstw.at[wid], dst_v)
    one16 = jnp.ones((16,), jnp.float32)
    z16 = _Z16()

    def fill(i, c):
        ones_v[pl.ds(i * 16, 16)] = one16
        return c
    lax.fori_loop(0, CH // 16, fill, 0)

    def zfill(i, c):
        zbuf[pl.ds(i * 16, 16)] = z16
        return c
    lax.fori_loop(0, RPT // 16, zfill, 0)

    rbase = sid * RPT
    pltpu.sync_copy(zbuf, acc.at[pl.ds(rbase, RPT)])
    plsc.subcore_barrier()

    def body(j, c):
        pltpu.sync_copy(ones_v, acc.at[dst_v.at[j]], add=True)
        return c
    lax.fori_loop(0, NCH, body, 0)

    plsc.subcore_barrier()
    pltpu.sync_copy(acc.at[pl.ds(rbase, RPT)], out.at[cid, pl.ds(rbase, RPT)])


@functools.cache
def _build_scat_sc():
    return functools.partial(
        pl.kernel,
        out_type=jax.ShapeDtypeStruct((NC, NPAD, H), jnp.float32),
        mesh=_mesh(),
        compiler_params=_SC_PARAMS,
        scratch_types=[
            pltpu.VMEM((NCH, CH), jnp.int32),
            pltpu.VMEM((NCH, CH), jnp.int32),
            pltpu.VMEM((2, CH, H), jnp.float32),
            pltpu.VMEM_SHARED((NPAD, H), jnp.float32),
            pltpu.VMEM_SHARED((NPAD, H), jnp.float32),
            pltpu.SemaphoreType.DMA,
            pltpu.SemaphoreType.DMA,
        ],
    )(_scat_sc_body)


def _scat_sc_body(y, srcw, dstw, out, src_v, dst_v, bufs, acc,
                  y_sh, sem_a, sem_b):
    cid = lax.axis_index("c")
    sid = lax.axis_index("s")
    wid = cid * NS + sid
    rbase = sid * RPT
    pltpu.sync_copy(y.at[pl.ds(rbase, RPT)], y_sh.at[pl.ds(rbase, RPT)])
    pltpu.sync_copy(srcw.at[wid], src_v)
    pltpu.sync_copy(dstw.at[wid], dst_v)

    z16 = _Z16()

    def zrow(i, c):
        for g in range(H // 16):
            bufs[0, i, pl.ds(g * 16, 16)] = z16
        return c
    lax.fori_loop(0, CH, zrow, 0)

    for r in range(RPT // CH):
        pltpu.sync_copy(bufs.at[0], acc.at[pl.ds(rbase + r * CH, CH)])
    plsc.subcore_barrier()

    def wait_a():
        pltpu.make_async_copy(y.at[pl.ds(0, CH)], bufs.at[0], sem_a).wait()

    def wait_b():
        pltpu.make_async_copy(y.at[pl.ds(0, CH)], bufs.at[1], sem_b).wait()

    pltpu.async_copy(y_sh.at[src_v.at[0]], bufs.at[0], sem_a)
    pltpu.async_copy(y_sh.at[src_v.at[1]], bufs.at[1], sem_b)

    def body(i, c):
        j = i * 2
        wait_a()
        pltpu.sync_copy(bufs.at[0], acc.at[dst_v.at[j]], add=True)
        pltpu.async_copy(y_sh.at[src_v.at[j + 2]], bufs.at[0], sem_a)
        wait_b()
        pltpu.sync_copy(bufs.at[1], acc.at[dst_v.at[j + 1]], add=True)
        pltpu.async_copy(y_sh.at[src_v.at[j + 3]], bufs.at[1], sem_b)
        return c
    lax.fori_loop(0, (NCH - 2) // 2, body, 0)

    wait_a()
    pltpu.sync_copy(bufs.at[0], acc.at[dst_v.at[NCH - 2]], add=True)
    wait_b()
    pltpu.sync_copy(bufs.at[1], acc.at[dst_v.at[NCH - 1]], add=True)

    plsc.subcore_barrier()
    pltpu.sync_copy(acc.at[pl.ds(rbase, RPT)], out.at[cid, pl.ds(rbase, RPT)])


_DOT = functools.partial(jnp.dot, preferred_element_type=jnp.float32)
H2 = 2 * H
F2 = 2 * F


def _half_masks():
    lane = lax.broadcasted_iota(jnp.int32, (1, H2), 1)
    m_l = (lane < H).astype(jnp.float32)
    return m_l, 1.0 - m_l


def _ln_lrelu_pair(t, g, be):
    m_l, m_r = _half_masks()
    t = jnp.where(t > 0, t, 0.01 * t)
    mu_l = jnp.sum(t * m_l, axis=1, keepdims=True) / H
    mu_r = jnp.sum(t * m_r, axis=1, keepdims=True) / H
    d = t - (mu_l * m_l + mu_r * m_r)
    v_l = jnp.sum(d * d * m_l, axis=1, keepdims=True) / H
    v_r = jnp.sum(d * d * m_r, axis=1, keepdims=True) / H
    v = v_l * m_l + v_r * m_r
    return d * lax.rsqrt(v + 1e-5) * g + be


def _pair_spec(w=H2):
    return pl.BlockSpec((PB, w), lambda i: (i, 0))


def _part_spec(w=H2):
    return pl.BlockSpec((NC, PB, w), lambda i: (0, i, 0))


def _full_spec(r, c):
    return pl.BlockSpec((r, c), lambda i: (0, 0))


def _dinv_body(d8, dinv_o):
    dinv_o[...] = lax.rsqrt(d8[0] + d8[1] + 1.0)


_dinvk = pl.pallas_call(
    _dinv_body,
    grid=(1,),
    in_specs=[pl.BlockSpec((NC, 8, NPAD // 8), lambda i: (0, 0, 0))],
    out_specs=pl.BlockSpec((8, NPAD // 8), lambda i: (0, 0)),
    out_shape=jax.ShapeDtypeStruct((8, NPAD // 8), jnp.float32),
)


def _tc0_body(x2, w1d, dinv, y1_o):
    y1_o[...] = dinv[...] * _DOT(x2[...], w1d[...])


_tc0 = pl.pallas_call(
    _tc0_body,
    grid=(NBLK,),
    in_specs=[_pair_spec(F2), _full_spec(F2, H2), _pair_spec()],
    out_specs=_pair_spec(),
    out_shape=jax.ShapeDtypeStruct((N2, H2), jnp.float32),
)


def _mid1_body(p, y, dinv, b, g, be, wd, h_o, yn_o):
    di = dinv[...]
    h = _ln_lrelu_pair(di * (p[0] + p[1] + y[...]) + b[...], g[...], be[...])
    h_o[...] = h
    yn_o[...] = di * _DOT(h, wd[...])


_mid1 = pl.pallas_call(
    _mid1_body,
    grid=(NBLK,),
    in_specs=[_part_spec(), _pair_spec(), _pair_spec(),
              _full_spec(1, H2), _full_spec(1, H2), _full_spec(1, H2),
              _full_spec(H2, H2)],
    out_specs=[_pair_spec(), _pair_spec()],
    out_shape=[jax.ShapeDtypeStruct((N2, H2), jnp.float32),
               jax.ShapeDtypeStruct((N2, H2), jnp.float32)],
)


def _mid2_body(p, y, dinv, b, g, be, wd, hprev, z_o, yn_o):
    di = dinv[...]
    h = _ln_lrelu_pair(di * (p[0] + p[1] + y[...]) + b[...], g[...], be[...])
    z = h + hprev[...]
    z_o[...] = z
    yn_o[...] = di * _DOT(z, wd[...])


_mid2 = pl.pallas_call(
    _mid2_body,
    grid=(NBLK,),
    in_specs=[_part_spec(), _pair_spec(), _pair_spec(),
              _full_spec(1, H2), _full_spec(1, H2), _full_spec(1, H2),
              _full_spec(H2, H2), _pair_spec()],
    out_specs=[_pair_spec(), _pair_spec()],
    out_shape=[jax.ShapeDtypeStruct((N2, H2), jnp.float32),
               jax.ShapeDtypeStruct((N2, H2), jnp.float32)],
)


def _mid3_body(p, y, dinv, b, g, be, y4_o):
    di = dinv[...]
    h = _ln_lrelu_pair(di * (p[0] + p[1] + y[...]) + b[...], g[...], be[...])
    y4_o[...] = di * h


_mid3 = pl.pallas_call(
    _mid3_body,
    grid=(NBLK,),
    in_specs=[_part_spec(), _pair_spec(), _pair_spec(),
              _full_spec(1, H2), _full_spec(1, H2), _full_spec(1, H2)],
    out_specs=_pair_spec(),
    out_shape=jax.ShapeDtypeStruct((N2, H2), jnp.float32),
)


def _fin_body(p, y, dinv, w4d, b4, xr_o):
    ahd = dinv[...] * (p[0] + p[1] + y[...])
    xr_o[...] = _DOT(ahd, w4d[...]) + b4[...]


_fin = pl.pallas_call(
    _fin_body,
    grid=(NBLK,),
    in_specs=[_part_spec(), _pair_spec(), _pair_spec(),
              _full_spec(H2, F2), _full_spec(1, F2)],
    out_specs=_pair_spec(F2),
    out_shape=jax.ShapeDtypeStruct((N2, F2), jnp.float32),
)


def _pair_vec(v):
    return jnp.concatenate([v, v])[None]


def _blockdiag(w):
    r, c = w.shape
    out = jnp.zeros((2 * r, 2 * c), jnp.float32)
    return out.at[:r, :c].set(w).at[r:, c:].set(w)


def kernel(x, edge_index, W1, b1, g1, be1, W2, b2, g2, be2, W3, b3, g3, be3,
           W4, b4):
    e = edge_index.shape[1]
    npad_e = EPAD - e
    pad_src = jnp.zeros((npad_e,), jnp.int32)
    pad_dst = N + (jnp.arange(npad_e, dtype=jnp.int32) % (NPAD - N))
    srcw = jnp.concatenate([edge_index[0], pad_src]).reshape(NW, NCH, CH)
    dstw = jnp.concatenate([edge_index[1], pad_dst]).reshape(NW, NCH, CH)

    x2 = x.reshape(N // 2, F2)
    w1d = _blockdiag(W1)
    w2d = _blockdiag(W2)
    w3d = _blockdiag(W3)
    w4d = _blockdiag(W4)
    b1p, g1p, be1p = _pair_vec(b1), _pair_vec(g1), _pair_vec(be1)
    b2p, g2p, be2p = _pair_vec(b2), _pair_vec(g2), _pair_vec(be2)
    b3p, g3p, be3p = _pair_vec(b3), _pair_vec(g3), _pair_vec(be3)
    b4p = _pair_vec(b4)

    deg_sc = _build_deg_sc()
    scat_sc = _build_scat_sc()

    degp = deg_sc(dstw)
    dinv8 = _dinvk(degp.reshape(NC, 8, NPAD // 8))
    dinv_b = jnp.broadcast_to(
        dinv8.reshape(NPAD)[:, None], (NPAD, H)).reshape(N2, H2)

    def scat(y_pairs):
        s = scat_sc(y_pairs.reshape(NPAD, H), srcw, dstw)
        return s.reshape(NC, N2, H2)

    y1 = _tc0(x2, w1d, dinv_b)
    s1 = scat(y1)
    h, y2 = _mid1(s1, y1, dinv_b, b1p, g1p, be1p, w2d)
    s2 = scat(y2)
    z, y3 = _mid2(s2, y2, dinv_b, b2p, g2p, be2p, w3d, h)
    s3 = scat(y3)
    y4 = _mid3(s3, y3, dinv_b, b3p, g3p, be3p)
    s4 = scat(y4)
    xr = _fin(s4, y4, dinv_b, w4d, b4p)
    return xr.reshape(NPAD, F)[:N], z.reshape(NPAD, H)[:N]


# speedup vs baseline: 35.5797x; 1.1016x over previous
import functools

import jax
import jax.numpy as jnp
from jax import lax
from jax.experimental import pallas as pl
from jax.experimental.pallas import tpu as pltpu
from jax.experimental.pallas import tpu_sc as plsc

N = 10000
F = 128
H = 64
NC = 2
NS = 16
NW = NC * NS
CH = 128
NCH = 80
EPAD = NW * NCH * CH
NPAD = 10240
RPT = NPAD // NS
RB = 1024
NBLK = NPAD // RB
N2 = NPAD // 2
PB = RB // 2

_Z16 = functools.partial(jnp.zeros, (16,), jnp.float32)


def _mesh():
    return plsc.VectorSubcoreMesh(
        core_axis_name="c", subcore_axis_name="s",
        num_cores=NC, num_subcores=NS)


_SC_PARAMS = pltpu.CompilerParams(use_tc_tiling_on_sc=False)


@functools.cache
def _build_deg_sc():
    return functools.partial(
        pl.kernel,
        out_type=jax.ShapeDtypeStruct((NC, NPAD), jnp.float32),
        mesh=_mesh(),
        compiler_params=_SC_PARAMS,
        scratch_types=[
            pltpu.VMEM((NCH, CH), jnp.int32),
            pltpu.VMEM((CH,), jnp.float32),
            pltpu.VMEM((RPT,), jnp.float32),
            pltpu.VMEM_SHARED((NPAD,), jnp.float32),
        ],
    )(_deg_sc_body)


def _deg_sc_body(dstw, out, dst_v, ones_v, zbuf, acc):
    cid = lax.axis_index("c")
    sid = lax.axis_index("s")
    wid = cid * NS + sid
    pltpu.sync_copy(dstw.at[wid], dst_v)
    one16 = jnp.ones((16,), jnp.float32)
    z16 = _Z16()

    def fill(i, c):
        ones_v[pl.ds(i * 16, 16)] = one16
        return c
    lax.fori_loop(0, CH // 16, fill, 0)

    def zfill(i, c):
        zbuf[pl.ds(i * 16, 16)] = z16
        return c
    lax.fori_loop(0, RPT // 16, zfill, 0)

    rbase = sid * RPT
    pltpu.sync_copy(zbuf, acc.at[pl.ds(rbase, RPT)])
    plsc.subcore_barrier()

    def body(j, c):
        pltpu.sync_copy(ones_v, acc.at[dst_v.at[j]], add=True)
        return c
    lax.fori_loop(0, NCH, body, 0)

    plsc.subcore_barrier()
    pltpu.sync_copy(acc.at[pl.ds(rbase, RPT)], out.at[cid, pl.ds(rbase, RPT)])


@functools.cache
def _build_scat_sc():
    return functools.partial(
        pl.kernel,
        out_type=jax.ShapeDtypeStruct((NC, NPAD, H), jnp.float32),
        mesh=_mesh(),
        compiler_params=_SC_PARAMS,
        scratch_types=[
            pltpu.VMEM((NCH, CH), jnp.int32),
            pltpu.VMEM((NCH, CH), jnp.int32),
            pltpu.VMEM((2, CH, H), jnp.float32),
            pltpu.VMEM_SHARED((NPAD, H), jnp.float32),
            pltpu.VMEM_SHARED((NPAD, H), jnp.float32),
            pltpu.SemaphoreType.DMA,
            pltpu.SemaphoreType.DMA,
            pltpu.SemaphoreType.DMA,
        ],
    )(_scat_sc_body)


def _scat_sc_body(y, srcw, dstw, out, src_v, dst_v, bufs, acc,
                  y_sh, sem_a, sem_b, sem_c):
    cid = lax.axis_index("c")
    sid = lax.axis_index("s")
    wid = cid * NS + sid
    rbase = sid * RPT
    pltpu.async_copy(y.at[pl.ds(rbase, RPT)], y_sh.at[pl.ds(rbase, RPT)],
                     sem_a)
    pltpu.async_copy(srcw.at[wid], src_v, sem_b)
    pltpu.async_copy(dstw.at[wid], dst_v, sem_c)

    z16 = _Z16()

    def zrow(i, c):
        for g in range(H // 16):
            bufs[0, i, pl.ds(g * 16, 16)] = z16
        return c
    lax.fori_loop(0, CH, zrow, 0)

    for r in range(RPT // CH):
        pltpu.sync_copy(bufs.at[0], acc.at[pl.ds(rbase + r * CH, CH)])
    pltpu.make_async_copy(y.at[pl.ds(rbase, RPT)],
                          y_sh.at[pl.ds(rbase, RPT)], sem_a).wait()
    pltpu.make_async_copy(srcw.at[wid], src_v, sem_b).wait()
    pltpu.make_async_copy(dstw.at[wid], dst_v, sem_c).wait()
    plsc.subcore_barrier()

    def wait_a():
        pltpu.make_async_copy(y.at[pl.ds(0, CH)], bufs.at[0], sem_a).wait()

    def wait_b():
        pltpu.make_async_copy(y.at[pl.ds(0, CH)], bufs.at[1], sem_b).wait()

    pltpu.async_copy(y_sh.at[src_v.at[0]], bufs.at[0], sem_a)
    pltpu.async_copy(y_sh.at[src_v.at[1]], bufs.at[1], sem_b)

    def body(i, c):
        j = i * 2
        wait_a()
        pltpu.sync_copy(bufs.at[0], acc.at[dst_v.at[j]], add=True)
        pltpu.async_copy(y_sh.at[src_v.at[j + 2]], bufs.at[0], sem_a)
        wait_b()
        pltpu.sync_copy(bufs.at[1], acc.at[dst_v.at[j + 1]], add=True)
        pltpu.async_copy(y_sh.at[src_v.at[j + 3]], bufs.at[1], sem_b)
        return c
    lax.fori_loop(0, (NCH - 2) // 2, body, 0)

    wait_a()
    pltpu.sync_copy(bufs.at[0], acc.at[dst_v.at[NCH - 2]], add=True)
    wait_b()
    pltpu.sync_copy(bufs.at[1], acc.at[dst_v.at[NCH - 1]], add=True)

    plsc.subcore_barrier()
    pltpu.sync_copy(acc.at[pl.ds(rbase, RPT)], out.at[cid, pl.ds(rbase, RPT)])


_DOT = functools.partial(jnp.dot, preferred_element_type=jnp.float32)
H2 = 2 * H
F2 = 2 * F


def _half_masks():
    lane = lax.broadcasted_iota(jnp.int32, (1, H2), 1)
    m_l = (lane < H).astype(jnp.float32)
    return m_l, 1.0 - m_l


def _ln_lrelu_pair(t, g, be):
    m_l, m_r = _half_masks()
    t = jnp.where(t > 0, t, 0.01 * t)
    mu_l = jnp.sum(t * m_l, axis=1, keepdims=True) / H
    mu_r = jnp.sum(t * m_r, axis=1, keepdims=True) / H
    d = t - (mu_l * m_l + mu_r * m_r)
    v_l = jnp.sum(d * d * m_l, axis=1, keepdims=True) / H
    v_r = jnp.sum(d * d * m_r, axis=1, keepdims=True) / H
    v = v_l * m_l + v_r * m_r
    return d * lax.rsqrt(v + 1e-5) * g + be


def _pair_spec(w=H2):
    return pl.BlockSpec((PB, w), lambda i: (i, 0))


def _part_spec(w=H2):
    return pl.BlockSpec((NC, PB, w), lambda i: (0, i, 0))


def _full_spec(r, c):
    return pl.BlockSpec((r, c), lambda i: (0, 0))


def _dinv_body(d8, dinv_o):
    dinv_o[...] = lax.rsqrt(d8[0] + d8[1] + 1.0)


_dinvk = pl.pallas_call(
    _dinv_body,
    grid=(1,),
    in_specs=[pl.BlockSpec((NC, 8, NPAD // 8), lambda i: (0, 0, 0))],
    out_specs=pl.BlockSpec((8, NPAD // 8), lambda i: (0, 0)),
    out_shape=jax.ShapeDtypeStruct((8, NPAD // 8), jnp.float32),
)


def _tc0_body(x2, w1d, dinv, y1_o):
    y1_o[...] = dinv[...] * _DOT(x2[...], w1d[...])


_tc0 = pl.pallas_call(
    _tc0_body,
    grid=(NBLK,),
    in_specs=[_pair_spec(F2), _full_spec(F2, H2), _pair_spec()],
    out_specs=_pair_spec(),
    out_shape=jax.ShapeDtypeStruct((N2, H2), jnp.float32),
)


def _mid1_body(p, y, dinv, b, g, be, wd, h_o, yn_o):
    di = dinv[...]
    h = _ln_lrelu_pair(di * (p[0] + p[1] + y[...]) + b[...], g[...], be[...])
    h_o[...] = h
    yn_o[...] = di * _DOT(h, wd[...])


_mid1 = pl.pallas_call(
    _mid1_body,
    grid=(NBLK,),
    in_specs=[_part_spec(), _pair_spec(), _pair_spec(),
              _full_spec(1, H2), _full_spec(1, H2), _full_spec(1, H2),
              _full_spec(H2, H2)],
    out_specs=[_pair_spec(), _pair_spec()],
    out_shape=[jax.ShapeDtypeStruct((N2, H2), jnp.float32),
               jax.ShapeDtypeStruct((N2, H2), jnp.float32)],
)


def _mid2_body(p, y, dinv, b, g, be, wd, hprev, z_o, yn_o):
    di = dinv[...]
    h = _ln_lrelu_pair(di * (p[0] + p[1] + y[...]) + b[...], g[...], be[...])
    z = h + hprev[...]
    z_o[...] = z
    yn_o[...] = di * _DOT(z, wd[...])


_mid2 = pl.pallas_call(
    _mid2_body,
    grid=(NBLK,),
    in_specs=[_part_spec(), _pair_spec(), _pair_spec(),
              _full_spec(1, H2), _full_spec(1, H2), _full_spec(1, H2),
              _full_spec(H2, H2), _pair_spec()],
    out_specs=[_pair_spec(), _pair_spec()],
    out_shape=[jax.ShapeDtypeStruct((N2, H2), jnp.float32),
               jax.ShapeDtypeStruct((N2, H2), jnp.float32)],
)


def _mid3_body(p, y, dinv, b, g, be, y4_o):
    di = dinv[...]
    h = _ln_lrelu_pair(di * (p[0] + p[1] + y[...]) + b[...], g[...], be[...])
    y4_o[...] = di * h


_mid3 = pl.pallas_call(
    _mid3_body,
    grid=(NBLK,),
    in_specs=[_part_spec(), _pair_spec(), _pair_spec(),
              _full_spec(1, H2), _full_spec(1, H2), _full_spec(1, H2)],
    out_specs=_pair_spec(),
    out_shape=jax.ShapeDtypeStruct((N2, H2), jnp.float32),
)


def _fin_body(p, y, dinv, w4d, b4, xr_o):
    ahd = dinv[...] * (p[0] + p[1] + y[...])
    xr_o[...] = _DOT(ahd, w4d[...]) + b4[...]


_fin = pl.pallas_call(
    _fin_body,
    grid=(NBLK,),
    in_specs=[_part_spec(), _pair_spec(), _pair_spec(),
              _full_spec(H2, F2), _full_spec(1, F2)],
    out_specs=_pair_spec(F2),
    out_shape=jax.ShapeDtypeStruct((N2, F2), jnp.float32),
)


def _pair_vec(v):
    return jnp.concatenate([v, v])[None]


def _blockdiag(w):
    r, c = w.shape
    out = jnp.zeros((2 * r, 2 * c), jnp.float32)
    return out.at[:r, :c].set(w).at[r:, c:].set(w)


def kernel(x, edge_index, W1, b1, g1, be1, W2, b2, g2, be2, W3, b3, g3, be3,
           W4, b4):
    e = edge_index.shape[1]
    npad_e = EPAD - e
    pad_src = jnp.zeros((npad_e,), jnp.int32)
    pad_dst = N + (jnp.arange(npad_e, dtype=jnp.int32) % (NPAD - N))
    srcw = jnp.concatenate([edge_index[0], pad_src]).reshape(NW, NCH, CH)
    dstw = jnp.concatenate([edge_index[1], pad_dst]).reshape(NW, NCH, CH)

    x2 = x.reshape(N // 2, F2)
    w1d = _blockdiag(W1)
    w2d = _blockdiag(W2)
    w3d = _blockdiag(W3)
    w4d = _blockdiag(W4)
    b1p, g1p, be1p = _pair_vec(b1), _pair_vec(g1), _pair_vec(be1)
    b2p, g2p, be2p = _pair_vec(b2), _pair_vec(g2), _pair_vec(be2)
    b3p, g3p, be3p = _pair_vec(b3), _pair_vec(g3), _pair_vec(be3)
    b4p = _pair_vec(b4)

    deg_sc = _build_deg_sc()
    scat_sc = _build_scat_sc()

    degp = deg_sc(dstw)
    dinv8 = _dinvk(degp.reshape(NC, 8, NPAD // 8))
    dinv_b = jnp.broadcast_to(
        dinv8.reshape(NPAD)[:, None], (NPAD, H)).reshape(N2, H2)

    def scat(y_pairs):
        s = scat_sc(y_pairs.reshape(NPAD, H), srcw, dstw)
        return s.reshape(NC, N2, H2)

    y1 = _tc0(x2, w1d, dinv_b)
    s1 = scat(y1)
    h, y2 = _mid1(s1, y1, dinv_b, b1p, g1p, be1p, w2d)
    s2 = scat(y2)
    z, y3 = _mid2(s2, y2, dinv_b, b2p, g2p, be2p, w3d, h)
    s3 = scat(y3)
    y4 = _mid3(s3, y3, dinv_b, b3p, g3p, be3p)
    s4 = scat(y4)
    xr = _fin(s4, y4, dinv_b, w4d, b4p)
    return xr.reshape(NPAD, F)[:N], z.reshape(NPAD, H)[:N]


# speedup vs baseline: 36.0608x; 1.0135x over previous
import functools

import jax
import jax.numpy as jnp
from jax import lax
from jax.experimental import pallas as pl
from jax.experimental.pallas import tpu as pltpu
from jax.experimental.pallas import tpu_sc as plsc

N = 10000
F = 128
H = 64
NC = 2
NS = 16
NW = NC * NS
CH = 128
NCH = 80
EPAD = NW * NCH * CH
NPAD = 10240
RPT = NPAD // NS
RB = 1024
NBLK = NPAD // RB
N2 = NPAD // 2
PB = RB // 2

_Z16 = functools.partial(jnp.zeros, (16,), jnp.float32)


def _mesh():
    return plsc.VectorSubcoreMesh(
        core_axis_name="c", subcore_axis_name="s",
        num_cores=NC, num_subcores=NS)


_SC_PARAMS = pltpu.CompilerParams(use_tc_tiling_on_sc=False)


@functools.cache
def _build_deg_sc():
    return functools.partial(
        pl.kernel,
        out_type=jax.ShapeDtypeStruct((NC, NPAD), jnp.float32),
        mesh=_mesh(),
        compiler_params=_SC_PARAMS,
        scratch_types=[
            pltpu.VMEM((NCH, CH), jnp.int32),
            pltpu.VMEM((CH,), jnp.float32),
            pltpu.VMEM((RPT,), jnp.float32),
            pltpu.VMEM_SHARED((NPAD,), jnp.float32),
        ],
    )(_deg_sc_body)


def _deg_sc_body(dstw, out, dst_v, ones_v, zbuf, acc):
    cid = lax.axis_index("c")
    sid = lax.axis_index("s")
    wid = cid * NS + sid
    pltpu.sync_copy(dstw.at[wid], dst_v)
    one16 = jnp.ones((16,), jnp.float32)
    z16 = _Z16()

    def fill(i, c):
        ones_v[pl.ds(i * 16, 16)] = one16
        return c
    lax.fori_loop(0, CH // 16, fill, 0)

    def zfill(i, c):
        zbuf[pl.ds(i * 16, 16)] = z16
        return c
    lax.fori_loop(0, RPT // 16, zfill, 0)

    rbase = sid * RPT
    pltpu.sync_copy(zbuf, acc.at[pl.ds(rbase, RPT)])
    plsc.subcore_barrier()

    def body(j, c):
        pltpu.sync_copy(ones_v, acc.at[dst_v.at[j]], add=True)
        return c
    lax.fori_loop(0, NCH, body, 0)

    plsc.subcore_barrier()
    pltpu.sync_copy(acc.at[pl.ds(rbase, RPT)], out.at[cid, pl.ds(rbase, RPT)])


@functools.cache
def _build_scat_sc():
    return functools.partial(
        pl.kernel,
        out_type=jax.ShapeDtypeStruct((NC, NPAD, H), jnp.float32),
        mesh=_mesh(),
        compiler_params=_SC_PARAMS,
        scratch_types=[
            pltpu.VMEM((NCH // 2, CH), jnp.int32),
            pltpu.VMEM((NCH // 2, CH), jnp.int32),
            pltpu.VMEM((4, CH, H), jnp.float32),
            pltpu.VMEM_SHARED((NPAD, H), jnp.float32),
            pltpu.VMEM_SHARED((NPAD, H), jnp.float32),
            [pltpu.SemaphoreType.DMA] * 4,
            [pltpu.SemaphoreType.DMA] * 4,
        ],
    )(_scat_sc_body)


def _scat_sc_body(y, srcw, dstw, out, src_v, dst_v, bufs, acc,
                  y_sh, gsem, ssem):
    cid = lax.axis_index("c")
    sid = lax.axis_index("s")
    wid = cid * NS + sid
    rbase = sid * RPT
    half = NCH // 2
    pltpu.async_copy(y.at[pl.ds(rbase, RPT)], y_sh.at[pl.ds(rbase, RPT)],
                     gsem[0])
    pltpu.async_copy(srcw.at[wid, pl.ds(0, half)], src_v, gsem[1])
    pltpu.async_copy(dstw.at[wid, pl.ds(0, half)], dst_v, gsem[2])

    z16 = _Z16()

    def zrow(i, c):
        for g in range(H // 16):
            bufs[0, i, pl.ds(g * 16, 16)] = z16
        return c
    lax.fori_loop(0, CH, zrow, 0)

    for r in range(RPT // CH):
        pltpu.sync_copy(bufs.at[0], acc.at[pl.ds(rbase + r * CH, CH)])
    pltpu.make_async_copy(y.at[pl.ds(rbase, RPT)],
                          y_sh.at[pl.ds(rbase, RPT)], gsem[0]).wait()
    pltpu.make_async_copy(srcw.at[wid, pl.ds(0, half)], src_v,
                          gsem[1]).wait()
    pltpu.make_async_copy(dstw.at[wid, pl.ds(0, half)], dst_v,
                          gsem[2]).wait()
    plsc.subcore_barrier()

    def start_g(k, jl):
        pltpu.async_copy(y_sh.at[src_v.at[jl]], bufs.at[k], gsem[k])

    def wait_g(k):
        pltpu.make_async_copy(y.at[pl.ds(0, CH)], bufs.at[k],
                              gsem[k]).wait()

    def start_s(k, jl):
        pltpu.async_copy(bufs.at[k], acc.at[dst_v.at[jl]], ssem[k],
                         add=True)

    def wait_s(k):
        pltpu.make_async_copy(bufs.at[k], acc.at[pl.ds(0, CH)],
                              ssem[k]).wait()

    for p in range(2):
        if p == 1:
            pltpu.sync_copy(srcw.at[wid, pl.ds(half, half)], src_v)
            pltpu.sync_copy(dstw.at[wid, pl.ds(half, half)], dst_v)
        for k in range(4):
            start_g(k, k)

        def body(i, c):
            j = i * 4
            wait_g(0)
            start_s(0, j)
            wait_g(1)
            start_s(1, j + 1)
            wait_s(0)
            start_g(0, j + 4)
            wait_s(1)
            start_g(1, j + 5)
            wait_g(2)
            start_s(2, j + 2)
            wait_g(3)
            start_s(3, j + 3)
            wait_s(2)
            start_g(2, j + 6)
            wait_s(3)
            start_g(3, j + 7)
            return c
        lax.fori_loop(0, half // 4 - 1, body, 0)

        jt = half - 4
        for k in range(4):
            wait_g(k)
            start_s(k, jt + k)
        for k in range(4):
            wait_s(k)

    plsc.subcore_barrier()
    pltpu.sync_copy(acc.at[pl.ds(rbase, RPT)], out.at[cid, pl.ds(rbase, RPT)])


_DOT = functools.partial(jnp.dot, preferred_element_type=jnp.float32)
H2 = 2 * H
F2 = 2 * F


def _half_masks():
    lane = lax.broadcasted_iota(jnp.int32, (1, H2), 1)
    m_l = (lane < H).astype(jnp.float32)
    return m_l, 1.0 - m_l


def _ln_lrelu_pair(t, g, be):
    m_l, m_r = _half_masks()
    t = jnp.where(t > 0, t, 0.01 * t)
    mu_l = jnp.sum(t * m_l, axis=1, keepdims=True) / H
    mu_r = jnp.sum(t * m_r, axis=1, keepdims=True) / H
    d = t - (mu_l * m_l + mu_r * m_r)
    v_l = jnp.sum(d * d * m_l, axis=1, keepdims=True) / H
    v_r = jnp.sum(d * d * m_r, axis=1, keepdims=True) / H
    v = v_l * m_l + v_r * m_r
    return d * lax.rsqrt(v + 1e-5) * g + be


def _pair_spec(w=H2):
    return pl.BlockSpec((PB, w), lambda i: (i, 0))


def _part_spec(w=H2):
    return pl.BlockSpec((NC, PB, w), lambda i: (0, i, 0))


def _full_spec(r, c):
    return pl.BlockSpec((r, c), lambda i: (0, 0))


def _dinv_body(d8, dinv_o):
    dinv_o[...] = lax.rsqrt(d8[0] + d8[1] + 1.0)


_dinvk = pl.pallas_call(
    _dinv_body,
    grid=(1,),
    in_specs=[pl.BlockSpec((NC, 8, NPAD // 8), lambda i: (0, 0, 0))],
    out_specs=pl.BlockSpec((8, NPAD // 8), lambda i: (0, 0)),
    out_shape=jax.ShapeDtypeStruct((8, NPAD // 8), jnp.float32),
)


def _tc0_body(x2, w1d, dinv, y1_o):
    y1_o[...] = dinv[...] * _DOT(x2[...], w1d[...])


_tc0 = pl.pallas_call(
    _tc0_body,
    grid=(NBLK,),
    in_specs=[_pair_spec(F2), _full_spec(F2, H2), _pair_spec()],
    out_specs=_pair_spec(),
    out_shape=jax.ShapeDtypeStruct((N2, H2), jnp.float32),
)


def _mid1_body(p, y, dinv, b, g, be, wd, h_o, yn_o):
    di = dinv[...]
    h = _ln_lrelu_pair(di * (p[0] + p[1] + y[...]) + b[...], g[...], be[...])
    h_o[...] = h
    yn_o[...] = di * _DOT(h, wd[...])


_mid1 = pl.pallas_call(
    _mid1_body,
    grid=(NBLK,),
    in_specs=[_part_spec(), _pair_spec(), _pair_spec(),
              _full_spec(1, H2), _full_spec(1, H2), _full_spec(1, H2),
              _full_spec(H2, H2)],
    out_specs=[_pair_spec(), _pair_spec()],
    out_shape=[jax.ShapeDtypeStruct((N2, H2), jnp.float32),
               jax.ShapeDtypeStruct((N2, H2), jnp.float32)],
)


def _mid2_body(p, y, dinv, b, g, be, wd, hprev, z_o, yn_o):
    di = dinv[...]
    h = _ln_lrelu_pair(di * (p[0] + p[1] + y[...]) + b[...], g[...], be[...])
    z = h + hprev[...]
    z_o[...] = z
    yn_o[...] = di * _DOT(z, wd[...])


_mid2 = pl.pallas_call(
    _mid2_body,
    grid=(NBLK,),
    in_specs=[_part_spec(), _pair_spec(), _pair_spec(),
              _full_spec(1, H2), _full_spec(1, H2), _full_spec(1, H2),
              _full_spec(H2, H2), _pair_spec()],
    out_specs=[_pair_spec(), _pair_spec()],
    out_shape=[jax.ShapeDtypeStruct((N2, H2), jnp.float32),
               jax.ShapeDtypeStruct((N2, H2), jnp.float32)],
)


def _mid3_body(p, y, dinv, b, g, be, y4_o):
    di = dinv[...]
    h = _ln_lrelu_pair(di * (p[0] + p[1] + y[...]) + b[...], g[...], be[...])
    y4_o[...] = di * h


_mid3 = pl.pallas_call(
    _mid3_body,
    grid=(NBLK,),
    in_specs=[_part_spec(), _pair_spec(), _pair_spec(),
              _full_spec(1, H2), _full_spec(1, H2), _full_spec(1, H2)],
    out_specs=_pair_spec(),
    out_shape=jax.ShapeDtypeStruct((N2, H2), jnp.float32),
)


def _fin_body(p, y, dinv, w4d, b4, xr_o):
    ahd = dinv[...] * (p[0] + p[1] + y[...])
    xr_o[...] = _DOT(ahd, w4d[...]) + b4[...]


_fin = pl.pallas_call(
    _fin_body,
    grid=(NBLK,),
    in_specs=[_part_spec(), _pair_spec(), _pair_spec(),
              _full_spec(H2, F2), _full_spec(1, F2)],
    out_specs=_pair_spec(F2),
    out_shape=jax.ShapeDtypeStruct((N2, F2), jnp.float32),
)


def _pair_vec(v):
    return jnp.concatenate([v, v])[None]


def _blockdiag(w):
    r, c = w.shape
    out = jnp.zeros((2 * r, 2 * c), jnp.float32)
    return out.at[:r, :c].set(w).at[r:, c:].set(w)


def kernel(x, edge_index, W1, b1, g1, be1, W2, b2, g2, be2, W3, b3, g3, be3,
           W4, b4):
    e = edge_index.shape[1]
    npad_e = EPAD - e
    pad_src = jnp.zeros((npad_e,), jnp.int32)
    pad_dst = N + (jnp.arange(npad_e, dtype=jnp.int32) % (NPAD - N))
    srcw = jnp.concatenate([edge_index[0], pad_src]).reshape(NW, NCH, CH)
    dstw = jnp.concatenate([edge_index[1], pad_dst]).reshape(NW, NCH, CH)

    x2 = x.reshape(N // 2, F2)
    w1d = _blockdiag(W1)
    w2d = _blockdiag(W2)
    w3d = _blockdiag(W3)
    w4d = _blockdiag(W4)
    b1p, g1p, be1p = _pair_vec(b1), _pair_vec(g1), _pair_vec(be1)
    b2p, g2p, be2p = _pair_vec(b2), _pair_vec(g2), _pair_vec(be2)
    b3p, g3p, be3p = _pair_vec(b3), _pair_vec(g3), _pair_vec(be3)
    b4p = _pair_vec(b4)

    deg_sc = _build_deg_sc()
    scat_sc = _build_scat_sc()

    degp = deg_sc(dstw)
    dinv8 = _dinvk(degp.reshape(NC, 8, NPAD // 8))
    dinv_b = jnp.broadcast_to(
        dinv8.reshape(NPAD)[:, None], (NPAD, H)).reshape(N2, H2)

    def scat(y_pairs):
        s = scat_sc(y_pairs.reshape(NPAD, H), srcw, dstw)
        return s.reshape(NC, N2, H2)

    y1 = _tc0(x2, w1d, dinv_b)
    s1 = scat(y1)
    h, y2 = _mid1(s1, y1, dinv_b, b1p, g1p, be1p, w2d)
    s2 = scat(y2)
    z, y3 = _mid2(s2, y2, dinv_b, b2p, g2p, be2p, w3d, h)
    s3 = scat(y3)
    y4 = _mid3(s3, y3, dinv_b, b3p, g3p, be3p)
    s4 = scat(y4)
    xr = _fin(s4, y4, dinv_b, w4d, b4p)
    return xr.reshape(NPAD, F)[:N], z.reshape(NPAD, H)[:N]
